# baseline (speedup 1.0000x reference)
import functools

import jax
import jax.numpy as jnp
from jax import lax
from jax.experimental import pallas as pl
from jax.experimental.pallas import tpu as pltpu
from jax.experimental.pallas import tpu_sc as plsc

_N = 320000
_D = 128
_C = 4
_L = 16
_NC = 2
_NS = 16
_NW = _NC * _NS
_ROWS_W = _N // _NW
_CH = 80
_NCHUNK = _ROWS_W // _CH
_RING = 5


def _sc_partials(features, labels_i32):
    mesh = plsc.VectorSubcoreMesh(core_axis_name="c", subcore_axis_name="s")

    @functools.partial(
        pl.kernel,
        out_type=[
            jax.ShapeDtypeStruct((_NW * _C, _D), jnp.float32),
            jax.ShapeDtypeStruct((_NW * _C, _L), jnp.float32),
        ],
        mesh=mesh,
        scratch_types=[
            pltpu.VMEM((_RING, _CH, _D), jnp.float32),
            pltpu.VMEM((_ROWS_W,), jnp.int32),
            pltpu.VMEM((_C, _D), jnp.float32),
            pltpu.VMEM((_C, _L), jnp.float32),
            pltpu.SemaphoreType.DMA,
            pltpu.SemaphoreType.DMA,
            pltpu.SemaphoreType.DMA,
            pltpu.SemaphoreType.DMA,
            pltpu.SemaphoreType.DMA,
            pltpu.SemaphoreType.DMA,
        ],
    )
    def k(feat_hbm, lab_hbm, sums_hbm, cnt_hbm, fbuf, lab_all, acc,
          cnt, lsem, sem0, sem1, sem2, sem3, sem4):
        cid = lax.axis_index("c")
        sid = lax.axis_index("s")
        wid = cid * _NS + sid
        base = wid * _ROWS_W
        sems = (sem0, sem1, sem2, sem3, sem4)
        zero = jnp.zeros((_L,), jnp.float32)

        pltpu.async_copy(lab_hbm.at[pl.ds(base, _ROWS_W)], lab_all, lsem)
        for r in range(_C):
            for j in range(_D // _L):
                acc[r, pl.ds(j * _L, _L)] = zero
            cnt[r, :] = zero

        def issue_input(i, buf):
            pltpu.async_copy(feat_hbm.at[pl.ds(base + i * _CH, _CH)],
                             fbuf.at[buf], sems[buf])

        def wait_input(i, buf):
            pltpu.make_async_copy(feat_hbm.at[pl.ds(base + i * _CH, _CH)],
                                  fbuf.at[buf], sems[buf]).wait()

        def chunk_compute(i, buf):
            def grp(g, carry):
                lv = lab_all[pl.ds(i * _CH + g * _L, _L)]
                for r in range(_L):
                    lab = lv[r]
                    row = g * _L + r
                    for j in range(_D // _L):
                        sl = pl.ds(j * _L, _L)
                        plsc.addupdate(acc.at[lab, sl], fbuf[buf, row, sl])
                return carry
            lax.fori_loop(0, _CH // _L, grp, 0)

        for b in range(_RING):
            issue_input(b, b)
        pltpu.make_async_copy(lab_hbm.at[pl.ds(base, _ROWS_W)], lab_all,
                              lsem).wait()

        def outer(o, carry):
            for b in range(_RING):
                i = o * _RING + b
                wait_input(i, b)
                chunk_compute(i, b)
                issue_input(i + _RING, b)
            return carry

        lax.fori_loop(0, _NCHUNK // _RING - 1, outer, 0)
        for b in range(_RING):
            i = _NCHUNK - _RING + b
            wait_input(i, b)
            chunk_compute(i, b)

        def cgrp(g, carry):
            for h in range(5):
                lv = lab_all[pl.ds((g * 5 + h) * _L, _L)]
                for r in range(_C):
                    cnt[r, :] += jnp.where(lv == r, 1.0, 0.0)
            return carry
        lax.fori_loop(0, _ROWS_W // (_L * 5), cgrp, 0)

        pltpu.sync_copy(acc, sums_hbm.at[pl.ds(wid * _C, _C)])
        pltpu.sync_copy(cnt, cnt_hbm.at[pl.ds(wid * _C, _C)])

    return k(features, labels_i32)


def _tc_body(s_ref, c_ref, out_ref):
    tot = s_ref[0:_C, :]
    for w in range(1, _NW):
        tot += s_ref[w * _C:(w + 1) * _C, :]
    ctot = c_ref[0:_C, :]
    for w in range(1, _NW):
        ctot += c_ref[w * _C:(w + 1) * _C, :]
    for cl in range(_C):
        n_cl = jnp.sum(ctot[cl, :])
        mean = tot[cl, :] / jnp.maximum(n_cl, 1.0)
        nrm = jnp.sqrt(jnp.sum(mean * mean))
        out_ref[cl, :] = mean / jnp.maximum(nrm, 1e-12)


def _tc_finalize(sums, cnts):
    return pl.pallas_call(
        _tc_body,
        out_shape=jax.ShapeDtypeStruct((_C, _D), jnp.float32),
    )(sums, cnts)


def kernel(features, labels):
    sums, cnts = _sc_partials(features, labels.astype(jnp.int32))
    fea_center = _tc_finalize(sums, cnts)
    target = jnp.array([0, 1, 2, 3], dtype=jnp.int64)
    return (fea_center, target)


# speedup vs baseline: 2.9326x; 2.9326x over previous
import functools

import jax
import jax.numpy as jnp
from jax import lax
from jax.experimental import pallas as pl
from jax.experimental.pallas import tpu as pltpu
from jax.experimental.pallas import tpu_sc as plsc

_N = 320000
_D = 128
_C = 4
_L = 16
_NC = 2
_NS = 16
_NW = _NC * _NS
_ROWS_W = _N // _NW
_CH = 80
_NCHUNK = _ROWS_W // _CH
_RING = 5


def _sc_partials(features, labels_i32):
    mesh = plsc.VectorSubcoreMesh(core_axis_name="c", subcore_axis_name="s")

    @functools.partial(
        pl.kernel,
        out_type=[
            jax.ShapeDtypeStruct((_NW * _C, _D), jnp.float32),
            jax.ShapeDtypeStruct((_NW * _C, _L), jnp.float32),
        ],
        mesh=mesh,
        scratch_types=[
            pltpu.VMEM((_RING, _CH, _D), jnp.float32),
            pltpu.VMEM((_ROWS_W,), jnp.int32),
            pltpu.VMEM((_RING, _CH), jnp.int32),
            pltpu.VMEM((_C, _D), jnp.float32),
            pltpu.VMEM((_C, _L), jnp.float32),
            pltpu.VMEM_SHARED((_NS * _C, _D), jnp.float32),
            pltpu.SemaphoreType.DMA,
            pltpu.SemaphoreType.DMA,
            pltpu.SemaphoreType.DMA,
            pltpu.SemaphoreType.DMA,
            pltpu.SemaphoreType.DMA,
            pltpu.SemaphoreType.DMA,
            pltpu.SemaphoreType.DMA,
            pltpu.SemaphoreType.DMA,
            pltpu.SemaphoreType.DMA,
            pltpu.SemaphoreType.DMA,
            pltpu.SemaphoreType.DMA,
        ],
    )
    def k(feat_hbm, lab_hbm, sums_hbm, cnt_hbm, fbuf, lab_all, lidx, zbuf,
          cnt, shared_acc, lsem, i0, i1, i2, i3, i4, s0, s1, s2, s3, s4):
        cid = lax.axis_index("c")
        sid = lax.axis_index("s")
        wid = cid * _NS + sid
        base = wid * _ROWS_W
        bank = sid * _C
        isems = (i0, i1, i2, i3, i4)
        ssems = (s0, s1, s2, s3, s4)
        zero = jnp.zeros((_L,), jnp.float32)

        pltpu.async_copy(lab_hbm.at[pl.ds(base, _ROWS_W)], lab_all, lsem)
        for r in range(_C):
            for j in range(_D // _L):
                zbuf[r, pl.ds(j * _L, _L)] = zero
            cnt[r, :] = zero
        pltpu.sync_copy(zbuf, shared_acc.at[pl.ds(bank, _C)])
        pltpu.make_async_copy(lab_hbm.at[pl.ds(base, _ROWS_W)], lab_all,
                              lsem).wait()

        def issue_input(i, buf):
            pltpu.async_copy(feat_hbm.at[pl.ds(base + i * _CH, _CH)],
                             fbuf.at[buf], isems[buf])

        def wait_input(i, buf):
            pltpu.make_async_copy(feat_hbm.at[pl.ds(base + i * _CH, _CH)],
                                  fbuf.at[buf], isems[buf]).wait()

        def issue_scatter(i, buf):
            for g in range(_CH // _L):
                lidx[buf, pl.ds(g * _L, _L)] = (
                    lab_all[pl.ds(i * _CH + g * _L, _L)] + bank)
            pltpu.async_copy(fbuf.at[buf], shared_acc.at[lidx.at[buf]],
                             ssems[buf], add=True)

        def wait_scatter(i, buf):
            pltpu.make_async_copy(fbuf.at[buf], shared_acc.at[lidx.at[buf]],
                                  ssems[buf]).wait()

        issue_input(0, 0)

        def prep(j, carry):
            for g in range(5):
                lv = lab_all[pl.ds((j * 5 + g) * _L, _L)]
                for r in range(_C):
                    cnt[r, :] += jnp.where(lv == r, 1.0, 0.0)
            return carry
        lax.fori_loop(0, _ROWS_W // (_L * 5), prep, 0)

        for i in range(3):
            if i >= 2:
                wait_scatter(i - 2, (i - 2) % _RING)
            issue_input(i + 1, (i + 1) % _RING)
            wait_input(i, i % _RING)
            issue_scatter(i, i % _RING)

        def outer(o, carry):
            for b in range(_RING):
                i = 3 + o * _RING + b
                wait_scatter(i - 2, (1 + b) % _RING)
                issue_input(i + 1, (4 + b) % _RING)
                wait_input(i, (3 + b) % _RING)
                issue_scatter(i, (3 + b) % _RING)
            return carry
        lax.fori_loop(0, (_NCHUNK - _RING) // _RING, outer, 0)

        for i in range(_NCHUNK - 2, _NCHUNK):
            wait_scatter(i - 2, (i - 2) % _RING)
            if i + 1 < _NCHUNK:
                issue_input(i + 1, (i + 1) % _RING)
            wait_input(i, i % _RING)
            issue_scatter(i, i % _RING)
        for i in range(_NCHUNK - 2, _NCHUNK):
            wait_scatter(i, i % _RING)

        pltpu.sync_copy(shared_acc.at[pl.ds(bank, _C)],
                        sums_hbm.at[pl.ds(wid * _C, _C)])
        pltpu.sync_copy(cnt, cnt_hbm.at[pl.ds(wid * _C, _C)])

    return k(features, labels_i32)


def _tc_body(s_ref, c_ref, out_ref):
    tot = s_ref[0:_C, :]
    for w in range(1, _NW):
        tot += s_ref[w * _C:(w + 1) * _C, :]
    ctot = c_ref[0:_C, :]
    for w in range(1, _NW):
        ctot += c_ref[w * _C:(w + 1) * _C, :]
    for cl in range(_C):
        n_cl = jnp.sum(ctot[cl, :])
        mean = tot[cl, :] / jnp.maximum(n_cl, 1.0)
        nrm = jnp.sqrt(jnp.sum(mean * mean))
        out_ref[cl, :] = mean / jnp.maximum(nrm, 1e-12)


def _tc_finalize(sums, cnts):
    return pl.pallas_call(
        _tc_body,
        out_shape=jax.ShapeDtypeStruct((_C, _D), jnp.float32),
    )(sums, cnts)


def kernel(features, labels):
    sums, cnts = _sc_partials(features, labels.astype(jnp.int32))
    fea_center = _tc_finalize(sums, cnts)
    target = jnp.array([0, 1, 2, 3], dtype=jnp.int64)
    return (fea_center, target)


# speedup vs baseline: 3.6474x; 1.2438x over previous
import functools

import jax
import jax.numpy as jnp
from jax import lax
from jax.experimental import pallas as pl
from jax.experimental.pallas import tpu as pltpu
from jax.experimental.pallas import tpu_sc as plsc

_N = 320000
_D = 128
_C = 4
_L = 16
_NC = 2
_NS = 16
_NW = _NC * _NS

_N_SC = 153600
_N_TC = _N - _N_SC
_ROWS_W = _N_SC // _NW
_CH = 80
_NCHUNK = _ROWS_W // _CH
_RING = 5
_BLK = 3200
_TC_NBLK = _N_TC // _BLK


def _sc_partials(features, labels_i32):
    mesh = plsc.VectorSubcoreMesh(core_axis_name="c", subcore_axis_name="s")

    @functools.partial(
        pl.kernel,
        out_type=[
            jax.ShapeDtypeStruct((_NW * _C, _D), jnp.float32),
            jax.ShapeDtypeStruct((_NW * _C, _L), jnp.float32),
        ],
        mesh=mesh,
        scratch_types=[
            pltpu.VMEM((_RING, _CH, _D), jnp.float32),
            pltpu.VMEM((_ROWS_W,), jnp.int32),
            pltpu.VMEM((_RING, _CH), jnp.int32),
            pltpu.VMEM((_C, _D), jnp.float32),
            pltpu.VMEM((_C, _L), jnp.float32),
            pltpu.VMEM_SHARED((_NS * _C, _D), jnp.float32),
            pltpu.SemaphoreType.DMA,
            pltpu.SemaphoreType.DMA,
            pltpu.SemaphoreType.DMA,
            pltpu.SemaphoreType.DMA,
            pltpu.SemaphoreType.DMA,
            pltpu.SemaphoreType.DMA,
            pltpu.SemaphoreType.DMA,
            pltpu.SemaphoreType.DMA,
            pltpu.SemaphoreType.DMA,
            pltpu.SemaphoreType.DMA,
            pltpu.SemaphoreType.DMA,
        ],
    )
    def k(feat_hbm, lab_hbm, sums_hbm, cnt_hbm, fbuf, lab_all, lidx, zbuf,
          cnt, shared_acc, lsem, i0, i1, i2, i3, i4, s0, s1, s2, s3, s4):
        cid = lax.axis_index("c")
        sid = lax.axis_index("s")
        wid = cid * _NS + sid
        base = _N_TC + wid * _ROWS_W
        bank = sid * _C
        isems = (i0, i1, i2, i3, i4)
        ssems = (s0, s1, s2, s3, s4)
        zero = jnp.zeros((_L,), jnp.float32)

        pltpu.async_copy(lab_hbm.at[pl.ds(base, _ROWS_W)], lab_all, lsem)
        for r in range(_C):
            for j in range(_D // _L):
                zbuf[r, pl.ds(j * _L, _L)] = zero
            cnt[r, :] = zero
        pltpu.sync_copy(zbuf, shared_acc.at[pl.ds(bank, _C)])
        pltpu.make_async_copy(lab_hbm.at[pl.ds(base, _ROWS_W)], lab_all,
                              lsem).wait()

        def issue_input(i, buf):
            pltpu.async_copy(feat_hbm.at[pl.ds(base + i * _CH, _CH)],
                             fbuf.at[buf], isems[buf])

        def wait_input(i, buf):
            pltpu.make_async_copy(feat_hbm.at[pl.ds(base + i * _CH, _CH)],
                                  fbuf.at[buf], isems[buf]).wait()

        def issue_scatter(i, buf):
            for g in range(_CH // _L):
                lidx[buf, pl.ds(g * _L, _L)] = (
                    lab_all[pl.ds(i * _CH + g * _L, _L)] + bank)
            pltpu.async_copy(fbuf.at[buf], shared_acc.at[lidx.at[buf]],
                             ssems[buf], add=True)

        def wait_scatter(i, buf):
            pltpu.make_async_copy(fbuf.at[buf], shared_acc.at[lidx.at[buf]],
                                  ssems[buf]).wait()

        issue_input(0, 0)

        def prep(j, carry):
            for g in range(5):
                lv = lab_all[pl.ds((j * 5 + g) * _L, _L)]
                for r in range(_C):
                    cnt[r, :] += jnp.where(lv == r, 1.0, 0.0)
            return carry
        lax.fori_loop(0, _ROWS_W // (_L * 5), prep, 0)

        for i in range(3):
            if i >= 2:
                wait_scatter(i - 2, (i - 2) % _RING)
            issue_input(i + 1, (i + 1) % _RING)
            wait_input(i, i % _RING)
            issue_scatter(i, i % _RING)

        def outer(o, carry):
            for b in range(_RING):
                i = 3 + o * _RING + b
                wait_scatter(i - 2, (1 + b) % _RING)
                issue_input(i + 1, (4 + b) % _RING)
                wait_input(i, (3 + b) % _RING)
                issue_scatter(i, (3 + b) % _RING)
            return carry
        lax.fori_loop(0, (_NCHUNK - _RING) // _RING, outer, 0)

        for i in range(_NCHUNK - 2, _NCHUNK):
            wait_scatter(i - 2, (i - 2) % _RING)
            if i + 1 < _NCHUNK:
                issue_input(i + 1, (i + 1) % _RING)
            wait_input(i, i % _RING)
            issue_scatter(i, i % _RING)
        for i in range(_NCHUNK - 2, _NCHUNK):
            wait_scatter(i, i % _RING)

        pltpu.sync_copy(shared_acc.at[pl.ds(bank, _C)],
                        sums_hbm.at[pl.ds(wid * _C, _C)])
        pltpu.sync_copy(cnt, cnt_hbm.at[pl.ds(wid * _C, _C)])

    return k(features, labels_i32)


def _tc_part_body(lab_ref, feat_ref, sums_ref, cnt_ref, acc_ref, csm_ref):
    i = pl.program_id(0)

    @pl.when(i == 0)
    def _init():
        acc_ref[...] = jnp.zeros_like(acc_ref)
        for c in range(_C):
            csm_ref[0, c] = 0.0

    lab = lab_ref[0, 0, :]
    feat = feat_ref[...]
    lab_col = lab[:, None]
    for c in range(_C):
        acc_ref[c, :] += jnp.sum(jnp.where(lab_col == c, feat, 0.0), axis=0)
        csm_ref[0, c] += jnp.sum((lab == c).astype(jnp.float32))

    @pl.when(i == _TC_NBLK - 1)
    def _fin():
        sums_ref[...] = acc_ref[...]
        for c in range(_C):
            cnt_ref[c, :] = jnp.full((_D,), csm_ref[0, c], jnp.float32)


def _tc_partials(features, labels_i32):
    lab3 = labels_i32[:_N_TC].reshape(_TC_NBLK, 1, _BLK)
    return pl.pallas_call(
        _tc_part_body,
        grid=(_TC_NBLK,),
        in_specs=[
            pl.BlockSpec((1, 1, _BLK), lambda i: (i, 0, 0)),
            pl.BlockSpec((_BLK, _D), lambda i: (i, 0)),
        ],
        out_specs=[
            pl.BlockSpec((_C, _D), lambda i: (0, 0)),
            pl.BlockSpec((_C, _D), lambda i: (0, 0)),
        ],
        out_shape=[
            jax.ShapeDtypeStruct((_C, _D), jnp.float32),
            jax.ShapeDtypeStruct((_C, _D), jnp.float32),
        ],
        scratch_shapes=[
            pltpu.VMEM((_C, _D), jnp.float32),
            pltpu.SMEM((1, _C), jnp.float32),
        ],
    )(lab3, features)


def _fin_body(ss_ref, sc_ref, ts_ref, tcnt_ref, out_ref):
    tot = ts_ref[...]
    for w in range(_NW):
        tot += ss_ref[w * _C:(w + 1) * _C, :]
    ctot = sc_ref[0:_C, :]
    for w in range(1, _NW):
        ctot += sc_ref[w * _C:(w + 1) * _C, :]
    for cl in range(_C):
        n_cl = jnp.sum(ctot[cl, :]) + tcnt_ref[cl, 0]
        mean = tot[cl, :] / jnp.maximum(n_cl, 1.0)
        nrm = jnp.sqrt(jnp.sum(mean * mean))
        out_ref[cl, :] = mean / jnp.maximum(nrm, 1e-12)


def _finalize(sc_sums, sc_cnts, tc_sums, tc_cnt):
    return pl.pallas_call(
        _fin_body,
        out_shape=jax.ShapeDtypeStruct((_C, _D), jnp.float32),
    )(sc_sums, sc_cnts, tc_sums, tc_cnt)


def kernel(features, labels):
    labels_i32 = labels.astype(jnp.int32)
    sc_sums, sc_cnts = _sc_partials(features, labels_i32)
    tc_sums, tc_cnt = _tc_partials(features, labels_i32)
    fea_center = _finalize(sc_sums, sc_cnts, tc_sums, tc_cnt)
    target = jnp.array([0, 1, 2, 3], dtype=jnp.int64)
    return (fea_center, target)


# speedup vs baseline: 3.9583x; 1.0853x over previous
import functools

import jax
import jax.numpy as jnp
from jax import lax
from jax.experimental import pallas as pl
from jax.experimental.pallas import tpu as pltpu
from jax.experimental.pallas import tpu_sc as plsc

_N = 320000
_D = 128
_C = 4
_L = 16
_NC = 2
_NS = 16
_NW = _NC * _NS

_N_SC = 179200
_N_TC = _N - _N_SC
_ROWS_W = _N_SC // _NW
_CH = 80
_NCHUNK = _ROWS_W // _CH
_RING = 5
_BLK = 3200
_TC_NBLK = _N_TC // _BLK


def _sc_partials(features, labels_i32):
    mesh = plsc.VectorSubcoreMesh(core_axis_name="c", subcore_axis_name="s")

    @functools.partial(
        pl.kernel,
        out_type=[
            jax.ShapeDtypeStruct((_NW * _C, _D), jnp.float32),
            jax.ShapeDtypeStruct((_NW * _C, _L), jnp.float32),
        ],
        mesh=mesh,
        scratch_types=[
            pltpu.VMEM((_RING, _CH, _D), jnp.float32),
            pltpu.VMEM((_ROWS_W,), jnp.int32),
            pltpu.VMEM((_RING, _CH), jnp.int32),
            pltpu.VMEM((_C, _D), jnp.float32),
            pltpu.VMEM((_C, _L), jnp.float32),
            pltpu.VMEM_SHARED((_NS * _C, _D), jnp.float32),
            pltpu.SemaphoreType.DMA,
            pltpu.SemaphoreType.DMA,
            pltpu.SemaphoreType.DMA,
            pltpu.SemaphoreType.DMA,
            pltpu.SemaphoreType.DMA,
            pltpu.SemaphoreType.DMA,
            pltpu.SemaphoreType.DMA,
            pltpu.SemaphoreType.DMA,
            pltpu.SemaphoreType.DMA,
            pltpu.SemaphoreType.DMA,
            pltpu.SemaphoreType.DMA,
        ],
    )
    def k(feat_hbm, lab_hbm, sums_hbm, cnt_hbm, fbuf, lab_all, lidx, zbuf,
          cnt, shared_acc, lsem, i0, i1, i2, i3, i4, s0, s1, s2, s3, s4):
        cid = lax.axis_index("c")
        sid = lax.axis_index("s")
        wid = cid * _NS + sid
        base = _N_TC + wid * _ROWS_W
        bank = sid * _C
        isems = (i0, i1, i2, i3, i4)
        ssems = (s0, s1, s2, s3, s4)
        zero = jnp.zeros((_L,), jnp.float32)

        pltpu.async_copy(lab_hbm.at[pl.ds(base, _ROWS_W)], lab_all, lsem)
        for r in range(_C):
            for j in range(_D // _L):
                zbuf[r, pl.ds(j * _L, _L)] = zero
            cnt[r, :] = zero
        pltpu.sync_copy(zbuf, shared_acc.at[pl.ds(bank, _C)])
        pltpu.make_async_copy(lab_hbm.at[pl.ds(base, _ROWS_W)], lab_all,
                              lsem).wait()

        def issue_input(i, buf):
            pltpu.async_copy(feat_hbm.at[pl.ds(base + i * _CH, _CH)],
                             fbuf.at[buf], isems[buf])

        def wait_input(i, buf):
            pltpu.make_async_copy(feat_hbm.at[pl.ds(base + i * _CH, _CH)],
                                  fbuf.at[buf], isems[buf]).wait()

        def issue_scatter(i, buf):
            for g in range(_CH // _L):
                lidx[buf, pl.ds(g * _L, _L)] = (
                    lab_all[pl.ds(i * _CH + g * _L, _L)] + bank)
            pltpu.async_copy(fbuf.at[buf], shared_acc.at[lidx.at[buf]],
                             ssems[buf], add=True)

        def wait_scatter(i, buf):
            pltpu.make_async_copy(fbuf.at[buf], shared_acc.at[lidx.at[buf]],
                                  ssems[buf]).wait()

        issue_input(0, 0)

        def prep(j, carry):
            for g in range(5):
                lv = lab_all[pl.ds((j * 5 + g) * _L, _L)]
                for r in range(_C):
                    cnt[r, :] += jnp.where(lv == r, 1.0, 0.0)
            return carry
        lax.fori_loop(0, _ROWS_W // (_L * 5), prep, 0)

        for i in range(3):
            if i >= 2:
                wait_scatter(i - 2, (i - 2) % _RING)
            issue_input(i + 1, (i + 1) % _RING)
            wait_input(i, i % _RING)
            issue_scatter(i, i % _RING)

        def outer(o, carry):
            for b in range(_RING):
                i = 3 + o * _RING + b
                wait_scatter(i - 2, (1 + b) % _RING)
                issue_input(i + 1, (4 + b) % _RING)
                wait_input(i, (3 + b) % _RING)
                issue_scatter(i, (3 + b) % _RING)
            return carry
        lax.fori_loop(0, (_NCHUNK - _RING) // _RING, outer, 0)

        for i in range(_NCHUNK - 2, _NCHUNK):
            wait_scatter(i - 2, (i - 2) % _RING)
            if i + 1 < _NCHUNK:
                issue_input(i + 1, (i + 1) % _RING)
            wait_input(i, i % _RING)
            issue_scatter(i, i % _RING)
        for i in range(_NCHUNK - 2, _NCHUNK):
            wait_scatter(i, i % _RING)

        pltpu.sync_copy(shared_acc.at[pl.ds(bank, _C)],
                        sums_hbm.at[pl.ds(wid * _C, _C)])
        pltpu.sync_copy(cnt, cnt_hbm.at[pl.ds(wid * _C, _C)])

    return k(features, labels_i32)


def _tc_part_body(lab_ref, feat_ref, sums_ref, cnt_ref, acc_ref, csm_ref):
    i = pl.program_id(0)

    @pl.when(i == 0)
    def _init():
        acc_ref[...] = jnp.zeros_like(acc_ref)
        for c in range(_C):
            csm_ref[0, c] = 0.0

    lab = lab_ref[0, 0, :]
    feat = feat_ref[...]
    lab_col = lab[:, None]
    for c in range(_C):
        acc_ref[c, :] += jnp.sum(jnp.where(lab_col == c, feat, 0.0), axis=0)
        csm_ref[0, c] += jnp.sum((lab == c).astype(jnp.float32))

    @pl.when(i == _TC_NBLK - 1)
    def _fin():
        sums_ref[...] = acc_ref[...]
        for c in range(_C):
            cnt_ref[c, :] = jnp.full((_D,), csm_ref[0, c], jnp.float32)


def _tc_partials(features, labels_i32):
    lab3 = labels_i32[:_N_TC].reshape(_TC_NBLK, 1, _BLK)
    return pl.pallas_call(
        _tc_part_body,
        grid=(_TC_NBLK,),
        in_specs=[
            pl.BlockSpec((1, 1, _BLK), lambda i: (i, 0, 0)),
            pl.BlockSpec((_BLK, _D), lambda i: (i, 0)),
        ],
        out_specs=[
            pl.BlockSpec((_C, _D), lambda i: (0, 0)),
            pl.BlockSpec((_C, _D), lambda i: (0, 0)),
        ],
        out_shape=[
            jax.ShapeDtypeStruct((_C, _D), jnp.float32),
            jax.ShapeDtypeStruct((_C, _D), jnp.float32),
        ],
        scratch_shapes=[
            pltpu.VMEM((_C, _D), jnp.float32),
            pltpu.SMEM((1, _C), jnp.float32),
        ],
    )(lab3, features)


def _fin_body(ss_ref, sc_ref, ts_ref, tcnt_ref, out_ref):
    tot = ts_ref[...]
    for w in range(_NW):
        tot += ss_ref[w * _C:(w + 1) * _C, :]
    ctot = sc_ref[0:_C, :]
    for w in range(1, _NW):
        ctot += sc_ref[w * _C:(w + 1) * _C, :]
    for cl in range(_C):
        n_cl = jnp.sum(ctot[cl, :]) + tcnt_ref[cl, 0]
        mean = tot[cl, :] / jnp.maximum(n_cl, 1.0)
        nrm = jnp.sqrt(jnp.sum(mean * mean))
        out_ref[cl, :] = mean / jnp.maximum(nrm, 1e-12)


def _finalize(sc_sums, sc_cnts, tc_sums, tc_cnt):
    return pl.pallas_call(
        _fin_body,
        out_shape=jax.ShapeDtypeStruct((_C, _D), jnp.float32),
    )(sc_sums, sc_cnts, tc_sums, tc_cnt)


def kernel(features, labels):
    labels_i32 = labels.astype(jnp.int32)
    sc_sums, sc_cnts = _sc_partials(features, labels_i32)
    tc_sums, tc_cnt = _tc_partials(features, labels_i32)
    fea_center = _finalize(sc_sums, sc_cnts, tc_sums, tc_cnt)
    target = jnp.array([0, 1, 2, 3], dtype=jnp.int64)
    return (fea_center, target)
